# Initial kernel scaffold; baseline (speedup 1.0000x reference)
import jax
import jax.numpy as jnp
from jax.experimental import pallas as pl


def kernel(user_id, sex, age_group, occupation, target_movie_id, sequence_movie_ids, sequence_ratings, user_id_table, sex_table, age_group_table, occupation_table, movie_table, genre_table, proc_W, proc_b, pos_table):
    raise NotImplementedError("write your pallas kernel here")


# speedup vs baseline: 1.4965x; 1.4965x over previous
import functools

import jax
import jax.numpy as jnp
from jax import lax
from jax.experimental import pallas as pl
from jax.experimental.pallas import tpu as pltpu
from jax.experimental.pallas import tpu_sc as plsc

DM = 316
DPAD = 320
NG = 18


def _proc_body(m_ref, g_ref, w1_ref, w2_ref, b_ref, u_ref, out_ref, uout_ref):
    acc = jnp.dot(m_ref[...], w1_ref[...], preferred_element_type=jnp.float32)
    acc = acc + jnp.dot(g_ref[...], w2_ref[...], preferred_element_type=jnp.float32)
    out_ref[...] = jnp.maximum(acc + b_ref[...], 0.0)
    uout_ref[:, :DM] = u_ref[...]
    uout_ref[:, DM:] = jnp.zeros_like(uout_ref[:, DM:])


def _precompute(movie_table, genre_table, w1, w2, bpad, user_table):
    v = movie_table.shape[0]
    rb = 800
    return pl.pallas_call(
        _proc_body,
        grid=(v // rb,),
        in_specs=[
            pl.BlockSpec((rb, DM), lambda i: (i, 0)),
            pl.BlockSpec((rb, NG), lambda i: (i, 0)),
            pl.BlockSpec((DM, DPAD), lambda i: (0, 0)),
            pl.BlockSpec((NG, DPAD), lambda i: (0, 0)),
            pl.BlockSpec((1, DPAD), lambda i: (0, 0)),
            pl.BlockSpec((rb, DM), lambda i: (i, 0)),
        ],
        out_specs=[
            pl.BlockSpec((rb, DPAD), lambda i: (i, 0)),
            pl.BlockSpec((rb, DPAD), lambda i: (i, 0)),
        ],
        out_shape=[
            jax.ShapeDtypeStruct((v, DPAD), jnp.float32),
            jax.ShapeDtypeStruct((user_table.shape[0], DPAD), jnp.float32),
        ],
    )(movie_table, genre_table, w1, w2, bpad, user_table)


def _sc_gather(p_tab, idx_all, user_tab, user_id):
    nc, ns = 2, 16
    nw = nc * ns
    n = idx_all.shape[0]
    per_w = n // nw
    ch = 128
    n_ch = per_w // ch
    b = user_id.shape[0]
    u_per_w = b // nw
    mesh = plsc.VectorSubcoreMesh(core_axis_name="c", subcore_axis_name="s", num_cores=nc, num_subcores=ns)

    @functools.partial(
        pl.kernel,
        mesh=mesh,
        compiler_params=pltpu.CompilerParams(use_tc_tiling_on_sc=False),
        out_type=[
            jax.ShapeDtypeStruct((n, DPAD), jnp.float32),
            jax.ShapeDtypeStruct((b, DPAD), jnp.float32),
        ],
        scratch_types=[
            pltpu.VMEM((ch,), jnp.int32),
            pltpu.VMEM((ch, DPAD), jnp.float32),
            pltpu.VMEM((u_per_w,), jnp.int32),
            pltpu.VMEM((u_per_w, DPAD), jnp.float32),
            pltpu.SemaphoreType.DMA,
        ],
    )
    def k(p_hbm, idx_hbm, utab_hbm, uid_hbm, g_hbm, u_hbm,
          idx_v, rows_v, uidx_v, urows_v, sem):
        wid = lax.axis_index("s") * nc + lax.axis_index("c")
        base = wid * per_w

        def body(c, carry):
            off = base + c * ch
            pltpu.sync_copy(idx_hbm.at[pl.ds(off, ch)], idx_v)
            pltpu.async_copy(p_hbm.at[idx_v], rows_v, sem).wait()
            pltpu.sync_copy(rows_v, g_hbm.at[pl.ds(off, ch)])
            return carry

        lax.fori_loop(0, n_ch, body, 0)

        ub = wid * u_per_w
        pltpu.sync_copy(uid_hbm.at[pl.ds(ub, u_per_w)], uidx_v)
        pltpu.async_copy(utab_hbm.at[uidx_v], urows_v, sem).wait()
        pltpu.sync_copy(urows_v, u_hbm.at[pl.ds(ub, u_per_w)])

    return k(p_tab, idx_all, user_tab, user_id)


def _finish_body(g_ref, r_ref, p_ref, u_ref, sx_ref, ag_ref, oc_ref,
                 st_ref, at_ref, ot_ref, out1_ref, out2_ref):
    g = g_ref[...][:, :, :DM]
    pos = p_ref[...]
    r = r_ref[...]
    out1_ref[...] = (g + pos[None, :, :]) * r[:, :, None]

    rb = u_ref.shape[0]

    def onehot_emb(x_ref, tab_ref, nv):
        x = x_ref[...]
        i = lax.broadcasted_iota(jnp.int32, (rb, nv), 1).astype(jnp.float32)
        oh = (x == i).astype(jnp.float32)
        return jnp.dot(oh, tab_ref[...], preferred_element_type=jnp.float32)

    e_s = onehot_emb(sx_ref, st_ref, 2)
    e_a = onehot_emb(ag_ref, at_ref, 7)
    e_o = onehot_emb(oc_ref, ot_ref, 21)
    out2_ref[...] = jnp.concatenate([u_ref[...][:, :DM], e_s, e_a, e_o], axis=1)


def _finish(g3, ratings_ext, pos_ext, u, sex_f, age_f, occ_f,
            sex_table, age_table, occ_table):
    b, s, _ = g3.shape
    rb = 32
    return pl.pallas_call(
        _finish_body,
        grid=(b // rb,),
        in_specs=[
            pl.BlockSpec((rb, s, DPAD), lambda i: (i, 0, 0)),
            pl.BlockSpec((rb, s), lambda i: (i, 0)),
            pl.BlockSpec((s, DM), lambda i: (0, 0)),
            pl.BlockSpec((rb, DPAD), lambda i: (i, 0)),
            pl.BlockSpec((rb, 1), lambda i: (i, 0)),
            pl.BlockSpec((rb, 1), lambda i: (i, 0)),
            pl.BlockSpec((rb, 1), lambda i: (i, 0)),
            pl.BlockSpec((2, 1), lambda i: (0, 0)),
            pl.BlockSpec((7, 2), lambda i: (0, 0)),
            pl.BlockSpec((21, 4), lambda i: (0, 0)),
        ],
        out_specs=[
            pl.BlockSpec((rb, s, DM), lambda i: (i, 0, 0)),
            pl.BlockSpec((rb, DM + 7), lambda i: (i, 0)),
        ],
        out_shape=[
            jax.ShapeDtypeStruct((b, s, DM), jnp.float32),
            jax.ShapeDtypeStruct((b, DM + 7), jnp.float32),
        ],
    )(g3, ratings_ext, pos_ext, u, sex_f, age_f, occ_f,
      sex_table, age_table, occ_table)


def kernel(user_id, sex, age_group, occupation, target_movie_id, sequence_movie_ids,
           sequence_ratings, user_id_table, sex_table, age_group_table, occupation_table,
           movie_table, genre_table, proc_W, proc_b, pos_table):
    b = user_id.shape[0]
    seq = pos_table.shape[0]

    w1 = jnp.pad(proc_W[:DM], ((0, 0), (0, DPAD - DM)))
    w2 = jnp.pad(proc_W[DM:], ((0, 0), (0, DPAD - DM)))
    bpad = jnp.pad(proc_b, (0, DPAD - DM)).reshape(1, DPAD)

    p_tab, u_tab = _precompute(movie_table, genre_table, w1, w2, bpad, user_id_table)

    idx_all = jnp.concatenate(
        [sequence_movie_ids, target_movie_id], axis=1
    ).astype(jnp.int32).reshape(b * seq)

    g, u = _sc_gather(p_tab, idx_all, u_tab, user_id.astype(jnp.int32))

    ratings_ext = jnp.concatenate(
        [sequence_ratings.astype(jnp.float32), jnp.ones((b, 1), jnp.float32)], axis=1)
    pos_ext = jnp.concatenate(
        [pos_table[:seq - 1], jnp.zeros((1, DM), jnp.float32)], axis=0)

    out1, out2 = _finish(
        g.reshape(b, seq, DPAD), ratings_ext, pos_ext, u,
        sex.astype(jnp.float32).reshape(b, 1),
        age_group.astype(jnp.float32).reshape(b, 1),
        occupation.astype(jnp.float32).reshape(b, 1),
        sex_table, age_group_table, occupation_table)
    return (out1, out2)


# speedup vs baseline: 1.8128x; 1.2114x over previous
import functools

import jax
import jax.numpy as jnp
from jax import lax
from jax.experimental import pallas as pl
from jax.experimental.pallas import tpu as pltpu
from jax.experimental.pallas import tpu_sc as plsc

DM = 316
DPAD = 384
NG = 18


def _proc_body(m_ref, g_ref, w1_ref, w2_ref, b_ref, u_ref, out_ref, uout_ref):
    acc = jnp.dot(m_ref[...], w1_ref[...], preferred_element_type=jnp.float32)
    acc = acc + jnp.dot(g_ref[...], w2_ref[...], preferred_element_type=jnp.float32)
    out_ref[...] = jnp.maximum(acc + b_ref[...], 0.0)
    uout_ref[:, :DM] = u_ref[...]
    uout_ref[:, DM:] = jnp.zeros_like(uout_ref[:, DM:])


def _precompute(movie_table, genre_table, w1, w2, bpad, user_table):
    v = movie_table.shape[0]
    rb = 800
    return pl.pallas_call(
        _proc_body,
        grid=(v // rb,),
        in_specs=[
            pl.BlockSpec((rb, DM), lambda i: (i, 0)),
            pl.BlockSpec((rb, NG), lambda i: (i, 0)),
            pl.BlockSpec((DM, DPAD), lambda i: (0, 0)),
            pl.BlockSpec((NG, DPAD), lambda i: (0, 0)),
            pl.BlockSpec((1, DPAD), lambda i: (0, 0)),
            pl.BlockSpec((rb, DM), lambda i: (i, 0)),
        ],
        out_specs=[
            pl.BlockSpec((rb, DPAD), lambda i: (i, 0)),
            pl.BlockSpec((rb, DPAD), lambda i: (i, 0)),
        ],
        out_shape=[
            jax.ShapeDtypeStruct((v, DPAD), jnp.float32),
            jax.ShapeDtypeStruct((user_table.shape[0], DPAD), jnp.float32),
        ],
    )(movie_table, genre_table, w1, w2, bpad, user_table)


def _sc_gather(p_tab, idx_all, user_tab, user_id):
    nc, ns = 2, 16
    nw = nc * ns
    n = idx_all.shape[0]
    per_w = n // nw
    ch = 128
    n_ch = per_w // ch
    b = user_id.shape[0]
    u_per_w = b // nw
    mesh = plsc.VectorSubcoreMesh(core_axis_name="c", subcore_axis_name="s", num_cores=nc, num_subcores=ns)

    @functools.partial(
        pl.kernel,
        mesh=mesh,
        out_type=[
            jax.ShapeDtypeStruct((n, DPAD), jnp.float32),
            jax.ShapeDtypeStruct((b, DPAD), jnp.float32),
        ],
        scratch_types=[
            pltpu.VMEM((ch,), jnp.int32),
            pltpu.VMEM((ch, DPAD), jnp.float32),
            pltpu.VMEM((u_per_w,), jnp.int32),
            pltpu.VMEM((u_per_w, DPAD), jnp.float32),
            pltpu.SemaphoreType.DMA,
        ],
    )
    def k(p_hbm, idx_hbm, utab_hbm, uid_hbm, g_hbm, u_hbm,
          idx_v, rows_v, uidx_v, urows_v, sem):
        wid = lax.axis_index("s") * nc + lax.axis_index("c")
        base = wid * per_w

        def body(c, carry):
            off = base + c * ch
            pltpu.sync_copy(idx_hbm.at[pl.ds(off, ch)], idx_v)
            pltpu.async_copy(p_hbm.at[idx_v], rows_v, sem).wait()
            pltpu.sync_copy(rows_v, g_hbm.at[pl.ds(off, ch)])
            return carry

        lax.fori_loop(0, n_ch, body, 0)

        ub = wid * u_per_w
        pltpu.sync_copy(uid_hbm.at[pl.ds(ub, u_per_w)], uidx_v)
        pltpu.async_copy(utab_hbm.at[uidx_v], urows_v, sem).wait()
        pltpu.sync_copy(urows_v, u_hbm.at[pl.ds(ub, u_per_w)])

    return k(p_tab, idx_all, user_tab, user_id)


def _finish_body(g_ref, r_ref, p_ref, u_ref, sx_ref, ag_ref, oc_ref,
                 st_ref, at_ref, ot_ref, out1_ref, out2_ref):
    g = g_ref[...][:, :, :DM]
    pos = p_ref[...]
    r = r_ref[...]
    out1_ref[...] = (g + pos[None, :, :]) * r[:, :, None]

    rb = u_ref.shape[0]

    def onehot_emb(x_ref, tab_ref, nv):
        x = x_ref[...]
        i = lax.broadcasted_iota(jnp.int32, (rb, nv), 1).astype(jnp.float32)
        oh = (x == i).astype(jnp.float32)
        return jnp.dot(oh, tab_ref[...], preferred_element_type=jnp.float32)

    e_s = onehot_emb(sx_ref, st_ref, 2)
    e_a = onehot_emb(ag_ref, at_ref, 7)
    e_o = onehot_emb(oc_ref, ot_ref, 21)
    out2_ref[...] = jnp.concatenate([u_ref[...][:, :DM], e_s, e_a, e_o], axis=1)


def _finish(g3, ratings_ext, pos_ext, u, sex_f, age_f, occ_f,
            sex_table, age_table, occ_table):
    b, s, _ = g3.shape
    rb = 32
    return pl.pallas_call(
        _finish_body,
        grid=(b // rb,),
        in_specs=[
            pl.BlockSpec((rb, s, DPAD), lambda i: (i, 0, 0)),
            pl.BlockSpec((rb, s), lambda i: (i, 0)),
            pl.BlockSpec((s, DM), lambda i: (0, 0)),
            pl.BlockSpec((rb, DPAD), lambda i: (i, 0)),
            pl.BlockSpec((rb, 1), lambda i: (i, 0)),
            pl.BlockSpec((rb, 1), lambda i: (i, 0)),
            pl.BlockSpec((rb, 1), lambda i: (i, 0)),
            pl.BlockSpec((2, 1), lambda i: (0, 0)),
            pl.BlockSpec((7, 2), lambda i: (0, 0)),
            pl.BlockSpec((21, 4), lambda i: (0, 0)),
        ],
        out_specs=[
            pl.BlockSpec((rb, s, DM), lambda i: (i, 0, 0)),
            pl.BlockSpec((rb, DM + 7), lambda i: (i, 0)),
        ],
        out_shape=[
            jax.ShapeDtypeStruct((b, s, DM), jnp.float32),
            jax.ShapeDtypeStruct((b, DM + 7), jnp.float32),
        ],
    )(g3, ratings_ext, pos_ext, u, sex_f, age_f, occ_f,
      sex_table, age_table, occ_table)


def kernel(user_id, sex, age_group, occupation, target_movie_id, sequence_movie_ids,
           sequence_ratings, user_id_table, sex_table, age_group_table, occupation_table,
           movie_table, genre_table, proc_W, proc_b, pos_table):
    b = user_id.shape[0]
    seq = pos_table.shape[0]

    w1 = jnp.pad(proc_W[:DM], ((0, 0), (0, DPAD - DM)))
    w2 = jnp.pad(proc_W[DM:], ((0, 0), (0, DPAD - DM)))
    bpad = jnp.pad(proc_b, (0, DPAD - DM)).reshape(1, DPAD)

    p_tab, u_tab = _precompute(movie_table, genre_table, w1, w2, bpad, user_id_table)

    idx_all = jnp.concatenate(
        [sequence_movie_ids, target_movie_id], axis=1
    ).astype(jnp.int32).reshape(b * seq)

    g, u = _sc_gather(p_tab, idx_all, u_tab, user_id.astype(jnp.int32))

    ratings_ext = jnp.concatenate(
        [sequence_ratings.astype(jnp.float32), jnp.ones((b, 1), jnp.float32)], axis=1)
    pos_ext = jnp.concatenate(
        [pos_table[:seq - 1], jnp.zeros((1, DM), jnp.float32)], axis=0)

    out1, out2 = _finish(
        g.reshape(b, seq, DPAD), ratings_ext, pos_ext, u,
        sex.astype(jnp.float32).reshape(b, 1),
        age_group.astype(jnp.float32).reshape(b, 1),
        occupation.astype(jnp.float32).reshape(b, 1),
        sex_table, age_group_table, occupation_table)
    return (out1, out2)
